# Initial kernel scaffold; baseline (speedup 1.0000x reference)
import jax
import jax.numpy as jnp
from jax.experimental import pallas as pl


def kernel(shape, color, pose, instance_id, t, pad, shape_table, color_table, inst_table, temp_table, pose_W, pose_b, sn_w, sn_b, cn_w, cn_b, pn_w, pn_b, in_w, in_b, tn_w, tn_b):
    raise NotImplementedError("write your pallas kernel here")


# speedup vs baseline: 3.7853x; 3.7853x over previous
import functools

import jax
import jax.numpy as jnp
import numpy as np
from jax import lax
from jax.experimental import pallas as pl
from jax.experimental.pallas import tpu as pltpu
from jax.experimental.pallas import tpu_sc as plsc

S, B, C = 200, 1024, 64
TOK = S * B
TSCALE = 0.005
EPS = 1e-5

NC, NS = 2, 16
NW = NC * NS
GRP = 128
GPW = TOK // (NW * GRP)
TPW = TOK // NW


def _ln_rows_body(t_ref, w_ref, b_ref, o_ref):
    x = t_ref[...]
    mu = jnp.mean(x, axis=-1, keepdims=True)
    var = jnp.mean((x - mu) ** 2, axis=-1, keepdims=True)
    o_ref[...] = (x - mu) * lax.rsqrt(var + EPS) * w_ref[...] + b_ref[...]


def _prenorm(table, w, b, block):
    n = table.shape[0]
    return pl.pallas_call(
        _ln_rows_body,
        grid=(n // block,),
        in_specs=[
            pl.BlockSpec((block, C), lambda i: (i, 0)),
            pl.BlockSpec((1, C), lambda i: (0, 0)),
            pl.BlockSpec((1, C), lambda i: (0, 0)),
        ],
        out_specs=pl.BlockSpec((block, C), lambda i: (i, 0)),
        out_shape=jax.ShapeDtypeStruct((n, C), jnp.float32),
    )(table, w.reshape(1, C), b.reshape(1, C))


def _make_gather_sum():
    mesh = plsc.VectorSubcoreMesh(core_axis_name="c", subcore_axis_name="s")

    @functools.partial(
        pl.kernel,
        mesh=mesh,
        compiler_params=pltpu.CompilerParams(use_tc_tiling_on_sc=False),
        out_type=jax.ShapeDtypeStruct((TOK, C), jnp.float32),
        scratch_types=[
            pltpu.VMEM((TPW,), jnp.int32),
            pltpu.VMEM((TPW,), jnp.int32),
            pltpu.VMEM((TPW,), jnp.int32),
            pltpu.VMEM((TPW,), jnp.int32),
            pltpu.VMEM((GRP, C), jnp.float32),
            pltpu.VMEM((GRP, C), jnp.float32),
            pltpu.VMEM((GRP, C), jnp.float32),
            pltpu.VMEM((GRP, C), jnp.float32),
            pltpu.SemaphoreType.DMA,
        ],
    )
    def gather_sum(st, ct, nt, tt, ixs, ixc, ixn, ixt, out,
                   vs, vc, vn, vt, rs, rc, rn, rt, sem):
        cid = lax.axis_index("c")
        sid = lax.axis_index("s")
        wid = sid * NC + cid
        tok0 = wid * TPW

        pltpu.sync_copy(ixs.at[pl.ds(tok0, TPW)], vs)
        pltpu.sync_copy(ixc.at[pl.ds(tok0, TPW)], vc)
        pltpu.sync_copy(ixn.at[pl.ds(tok0, TPW)], vn)
        pltpu.sync_copy(ixt.at[pl.ds(tok0, TPW)], vt)

        def group(g, carry):
            off = g * GRP
            tok = tok0 + off
            c1 = pltpu.async_copy(st.at[vs.at[pl.ds(off, GRP)]], rs, sem)
            c2 = pltpu.async_copy(ct.at[vc.at[pl.ds(off, GRP)]], rc, sem)
            c3 = pltpu.async_copy(nt.at[vn.at[pl.ds(off, GRP)]], rn, sem)
            c4 = pltpu.async_copy(tt.at[vt.at[pl.ds(off, GRP)]], rt, sem)
            c1.wait()
            c2.wait()
            c3.wait()
            c4.wait()

            def tok_body(j, cc):
                for q in range(C // 16):
                    sl = pl.ds(q * 16, 16)
                    rs[j, sl] = rs[j, sl] + rc[j, sl] + rn[j, sl] + rt[j, sl]
                return cc

            lax.fori_loop(0, GRP, tok_body, 0)
            pltpu.sync_copy(rs, out.at[pl.ds(tok, GRP)])
            return carry

        lax.fori_loop(0, GPW, group, 0)

    return gather_sum


_gather_sum = _make_gather_sum()


def _pose_body(p_ref, part_ref, W_ref, pb_ref, w_ref, b_ref, sc_ref, o_ref):
    x = p_ref[...] * sc_ref[...]
    pe = jnp.dot(x, W_ref[...], preferred_element_type=jnp.float32) + pb_ref[...]
    mu = jnp.mean(pe, axis=-1, keepdims=True)
    var = jnp.mean((pe - mu) ** 2, axis=-1, keepdims=True)
    o_ref[...] = ((pe - mu) * lax.rsqrt(var + EPS) * w_ref[...] + b_ref[...]
                  + part_ref[...])


_POSE_R = 2048


def _pose_add(pose_flat, partial, W16, pose_b, pn_w, pn_b, scale):
    return pl.pallas_call(
        _pose_body,
        grid=(TOK // _POSE_R,),
        in_specs=[
            pl.BlockSpec((_POSE_R, 16), lambda i: (i, 0)),
            pl.BlockSpec((_POSE_R, C), lambda i: (i, 0)),
            pl.BlockSpec((16, C), lambda i: (0, 0)),
            pl.BlockSpec((1, C), lambda i: (0, 0)),
            pl.BlockSpec((1, C), lambda i: (0, 0)),
            pl.BlockSpec((1, C), lambda i: (0, 0)),
            pl.BlockSpec((1, 16), lambda i: (0, 0)),
        ],
        out_specs=pl.BlockSpec((_POSE_R, C), lambda i: (i, 0)),
        out_shape=jax.ShapeDtypeStruct((TOK, C), jnp.float32),
    )(pose_flat, partial, W16, pose_b.reshape(1, C), pn_w.reshape(1, C),
      pn_b.reshape(1, C), scale)


_SCALE16 = np.ones((1, 16), np.float32)
_SCALE16[0, [3, 7, 11]] = TSCALE


def kernel(shape, color, pose, instance_id, t, pad,
           shape_table, color_table, inst_table, temp_table,
           pose_W, pose_b,
           sn_w, sn_b, cn_w, cn_b, pn_w, pn_b, in_w, in_b, tn_w, tn_b):
    del pad

    ixs = shape.reshape(TOK).astype(jnp.int32)
    ixc = color.reshape(TOK).astype(jnp.int32)
    ixn = instance_id.reshape(TOK).astype(jnp.int32)
    ixt = t.reshape(TOK).astype(jnp.int32)

    stn = _prenorm(shape_table, sn_w, sn_b, 1000)
    ctn = _prenorm(color_table, cn_w, cn_b, 1000)
    n_inst = inst_table.shape[0]
    inst_padded = jnp.concatenate(
        [inst_table, jnp.zeros((1024 - n_inst, C), jnp.float32)], axis=0)
    ntn = _prenorm(inst_padded, in_w, in_b, 1024)
    ttn = _prenorm(temp_table, tn_w, tn_b, 1024)

    partial = _gather_sum(stn, ctn, ntn, ttn, ixs, ixc, ixn, ixt)

    W16 = jnp.zeros((16, C), jnp.float32).at[:12].set(pose_W)
    pose_flat = pose.reshape(TOK, 16)
    out = _pose_add(pose_flat, partial, W16, pose_b, pn_w, pn_b,
                    jnp.asarray(_SCALE16))
    return out.reshape(S, B, C)
